# baseline (speedup 1.0000x reference)
import functools

import jax
import jax.numpy as jnp
from jax import lax
from jax.experimental import pallas as pl
from jax.experimental.pallas import tpu as pltpu
from jax.experimental.pallas import tpu_sc as plsc

B, U, NUM_UTYPE, NUM_UFEAT, EMB_DIM = 4096, 50, 1000000, 26, 32
BU = B * U

NC, NS = 2, 16
NW = NC * NS
ROWS_PER_W = BU // NW
CHUNK = 128
NCHUNK = ROWS_PER_W // CHUNK


def _sc_gather_body(table_hbm, idx_hbm, out_hbm, idx_v, rows_v, sem):
    wid = lax.axis_index("s") * NC + lax.axis_index("c")
    base = wid * ROWS_PER_W
    pltpu.sync_copy(idx_hbm.at[wid], idx_v)

    def chunk_body(j, carry):
        pltpu.async_copy(table_hbm.at[idx_v.at[j]], rows_v, sem).wait()
        pltpu.sync_copy(rows_v, out_hbm.at[pl.ds(base + j * CHUNK, CHUNK)])
        return carry

    lax.fori_loop(0, NCHUNK, chunk_body, 0)


@jax.jit
def _sc_gather(table, idx3):
    mesh = plsc.VectorSubcoreMesh(core_axis_name="c", subcore_axis_name="s")
    fn = pl.kernel(
        _sc_gather_body,
        out_type=jax.ShapeDtypeStruct((BU, EMB_DIM), jnp.float32),
        mesh=mesh,
        scratch_types=[
            pltpu.VMEM((NCHUNK, CHUNK), jnp.int32),
            pltpu.VMEM((CHUNK, EMB_DIM), jnp.float32),
            pltpu.SemaphoreType.DMA,
        ],
        compiler_params=pltpu.CompilerParams(use_tc_tiling_on_sc=False),
    )
    return fn(table, idx3)


ROWS_PER_BLK = 2048


def _tc_combine_body(gath_ref, u_ref, v_ref, g_ref, b_ref, out_ref):
    v = v_ref[...]
    scale = g_ref[0, 0] * lax.rsqrt(jnp.sum(v * v))
    w = v * scale
    mm = jax.lax.dot_general(
        u_ref[...], w,
        dimension_numbers=(((1,), (1,)), ((), ())),
        preferred_element_type=jnp.float32,
    )
    out_ref[...] = jnp.concatenate([gath_ref[...], mm + b_ref[...]], axis=1)


@jax.jit
def _tc_combine(gathered, ufeat2, V, g2, b2):
    grid = (BU // ROWS_PER_BLK,)
    return pl.pallas_call(
        _tc_combine_body,
        grid=grid,
        in_specs=[
            pl.BlockSpec((ROWS_PER_BLK, EMB_DIM), lambda i: (i, 0)),
            pl.BlockSpec((ROWS_PER_BLK, NUM_UFEAT), lambda i: (i, 0)),
            pl.BlockSpec((EMB_DIM, NUM_UFEAT), lambda i: (0, 0)),
            pl.BlockSpec((1, 1), lambda i: (0, 0), memory_space=pltpu.SMEM),
            pl.BlockSpec((1, EMB_DIM), lambda i: (0, 0)),
        ],
        out_specs=pl.BlockSpec((ROWS_PER_BLK, 2 * EMB_DIM), lambda i: (i, 0)),
        out_shape=jax.ShapeDtypeStruct((BU, 2 * EMB_DIM), jnp.float32),
    )(gathered, ufeat2, V, g2, b2)


def kernel(utype, ufeat, table, V, g, b):
    idx3 = utype.astype(jnp.int32).reshape(NW, NCHUNK, CHUNK)
    gathered = _sc_gather(table, idx3)
    g2 = jnp.reshape(g, (1, 1)).astype(jnp.float32)
    b2 = jnp.reshape(b, (1, EMB_DIM))
    out = _tc_combine(gathered, ufeat.reshape(BU, NUM_UFEAT), V, g2, b2)
    return out.reshape(B, U, 2 * EMB_DIM)


# speedup vs baseline: 1.4870x; 1.4870x over previous
import functools

import jax
import jax.numpy as jnp
from jax import lax
from jax.experimental import pallas as pl
from jax.experimental.pallas import tpu as pltpu
from jax.experimental.pallas import tpu_sc as plsc

B, U, NUM_UTYPE, NUM_UFEAT, EMB_DIM = 4096, 50, 1000000, 26, 32
BU = B * U

NC, NS = 2, 16
NW = NC * NS
LANES = 128
PACK = LANES // EMB_DIM
RM_ROWS = NUM_UTYPE // PACK


TBLK = 16384


def _tr_body(in_ref, out_ref):
    out_ref[...] = in_ref[...].T


@jax.jit
def _tc_transpose(tableT):
    grid = (pl.cdiv(NUM_UTYPE, TBLK),)
    return pl.pallas_call(
        _tr_body,
        grid=grid,
        in_specs=[pl.BlockSpec((EMB_DIM, TBLK), lambda k: (0, k))],
        out_specs=pl.BlockSpec((TBLK, EMB_DIM), lambda k: (k, 0)),
        out_shape=jax.ShapeDtypeStruct((NUM_UTYPE, EMB_DIM), jnp.float32),
    )(tableT)


ROWS_PER_W = BU // NW
CHUNK = 128
NCHUNK = ROWS_PER_W // CHUNK


def _sc_gather_body(table_hbm, idx_hbm, out_hbm, idx_v, rows0, rows1,
                    sem0, sem1):
    wid = lax.axis_index("s") * NC + lax.axis_index("c")
    base = wid * ROWS_PER_W
    pltpu.sync_copy(idx_hbm.at[wid], idx_v)

    def start(j, buf, sem):
        pltpu.make_async_copy(table_hbm.at[idx_v.at[j]], buf, sem).start()

    def drain(j, buf, sem):
        pltpu.make_async_copy(table_hbm.at[idx_v.at[j]], buf, sem).wait()
        pltpu.sync_copy(buf, out_hbm.at[pl.ds(base + j * CHUNK, CHUNK)])

    start(0, rows0, sem0)
    start(1, rows1, sem1)

    def body(h, carry):
        j0 = 2 * h
        drain(j0, rows0, sem0)

        @pl.when(j0 + 2 < NCHUNK)
        def _():
            start(j0 + 2, rows0, sem0)

        drain(j0 + 1, rows1, sem1)

        @pl.when(j0 + 3 < NCHUNK)
        def _():
            start(j0 + 3, rows1, sem1)

        return carry

    lax.fori_loop(0, NCHUNK // 2, body, 0)


@jax.jit
def _sc_gather(tableRM, idx3):
    mesh = plsc.VectorSubcoreMesh(core_axis_name="c", subcore_axis_name="s")
    fn = pl.kernel(
        _sc_gather_body,
        out_type=jax.ShapeDtypeStruct((BU, EMB_DIM), jnp.float32),
        mesh=mesh,
        scratch_types=[
            pltpu.VMEM((NCHUNK, CHUNK), jnp.int32),
            pltpu.VMEM((CHUNK, EMB_DIM), jnp.float32),
            pltpu.VMEM((CHUNK, EMB_DIM), jnp.float32),
            pltpu.SemaphoreType.DMA,
            pltpu.SemaphoreType.DMA,
        ],
        compiler_params=pltpu.CompilerParams(use_tc_tiling_on_sc=False),
    )
    return fn(tableRM, idx3)


CB = 256


def _cmb_body(g_ref, u_ref, v_ref, gn_ref, b_ref, out_ref):
    v = v_ref[...]
    scale = gn_ref[0, 0] * lax.rsqrt(jnp.sum(v * v))
    w = v * scale
    bias = b_ref[...]
    for u in range(U):
        x = u_ref[:, u, :]
        mm = jax.lax.dot_general(
            w, x, dimension_numbers=(((1,), (0,)), ((), ())),
            preferred_element_type=jnp.float32,
        ) + bias
        gt = g_ref[u].T
        out_ref[u] = jnp.concatenate([gt, mm], axis=0)


@jax.jit
def _tc_combine(gathered3, ufeatT, V, g2, bcol):
    return pl.pallas_call(
        _cmb_body,
        grid=(B // CB,),
        in_specs=[
            pl.BlockSpec((U, CB, EMB_DIM), lambda c: (0, c, 0)),
            pl.BlockSpec((NUM_UFEAT, U, CB), lambda c: (0, 0, c)),
            pl.BlockSpec((EMB_DIM, NUM_UFEAT), lambda c: (0, 0)),
            pl.BlockSpec((1, 1), lambda c: (0, 0), memory_space=pltpu.SMEM),
            pl.BlockSpec((EMB_DIM, 1), lambda c: (0, 0)),
        ],
        out_specs=pl.BlockSpec((U, 2 * EMB_DIM, CB), lambda c: (0, 0, c)),
        out_shape=jax.ShapeDtypeStruct((U, 2 * EMB_DIM, B), jnp.float32),
    )(gathered3, ufeatT, V, g2, bcol)


def kernel(utype, ufeat, table, V, g, b):
    tableT = table.T
    ufeatT = jnp.transpose(ufeat, (2, 1, 0))
    idx3 = utype.T.astype(jnp.int32).reshape(NW, NCHUNK, CHUNK)
    tableRM = _tc_transpose(tableT)
    gathered = _sc_gather(tableRM, idx3)
    g2 = jnp.reshape(g, (1, 1)).astype(jnp.float32)
    bcol = jnp.reshape(b, (EMB_DIM, 1))
    outT = _tc_combine(gathered.reshape(U, B, EMB_DIM), ufeatT, V, g2, bcol)
    return outT.transpose(2, 0, 1)


# speedup vs baseline: 2.0472x; 1.3767x over previous
import functools

import jax
import jax.numpy as jnp
from jax import lax
from jax.experimental import pallas as pl
from jax.experimental.pallas import tpu as pltpu
from jax.experimental.pallas import tpu_sc as plsc

B, U, NUM_UTYPE, NUM_UFEAT, EMB_DIM = 4096, 50, 1000000, 26, 32
BU = B * U

NC, NS = 2, 16
NW = NC * NS
LANES = 128
PACK = LANES // EMB_DIM
RM_ROWS = NUM_UTYPE // PACK


TBLK = 16384
QBLK = TBLK // PACK
NTBLK = pl.cdiv(NUM_UTYPE, TBLK)
RM_PAD_ROWS = NTBLK * QBLK


def _tr_body(in_ref, out_ref):
    x = in_ref[...]
    parts = [x[:, q * QBLK:(q + 1) * QBLK].T for q in range(PACK)]
    out_ref[...] = jnp.concatenate(parts, axis=1)


@jax.jit
def _tc_transpose(tableT):
    return pl.pallas_call(
        _tr_body,
        grid=(NTBLK,),
        in_specs=[pl.BlockSpec((EMB_DIM, TBLK), lambda k: (0, k))],
        out_specs=pl.BlockSpec((QBLK, LANES), lambda k: (k, 0)),
        out_shape=jax.ShapeDtypeStruct((RM_PAD_ROWS, LANES), jnp.float32),
    )(tableT)


ROWS_PER_W = BU // NW
CHUNK = 128
NCHUNK = ROWS_PER_W // CHUNK
OUT_RPW = ROWS_PER_W * EMB_DIM // LANES


def _sc_gather_body(table_hbm, idx_hbm, out_hbm, idx_v,
                    ridx0, ridx1, g0, g1, robuf, sem0, sem1):
    wid = lax.axis_index("s") * NC + lax.axis_index("c")
    pltpu.sync_copy(idx_hbm.at[pl.ds(wid * NCHUNK, NCHUNK)], idx_v)

    def start(j, ridx, buf, sem):
        for m in range(CHUNK // 16):
            iv = idx_v[j, pl.ds(m * 16, 16)]
            ridx[pl.ds(m * 16, 16)] = lax.shift_left(
                lax.shift_right_logical(iv, 14), 12) | (iv & 4095)
        pltpu.make_async_copy(table_hbm.at[ridx], buf, sem).start()

    def drain(j, ridx, buf, sem):
        pltpu.make_async_copy(table_hbm.at[ridx], buf, sem).wait()
        for m in range(CHUNK // 16):
            iv = idx_v[j, pl.ds(m * 16, 16)]
            ov = (lax.shift_right_logical(iv, 12) & 3) * EMB_DIM
            for k16 in range(16):
                k = m * 16 + k16
                off = ov[k16]
                r, c = k // PACK, (k % PACK) * EMB_DIM
                robuf[r, pl.ds(c, 16)] = buf[k, pl.ds(off, 16)]
                robuf[r, pl.ds(c + 16, 16)] = buf[k, pl.ds(off + 16, 16)]
        pltpu.sync_copy(
            robuf, out_hbm.at[pl.ds(wid * OUT_RPW + j * (CHUNK // PACK),
                                    CHUNK // PACK)])

    start(0, ridx0, g0, sem0)
    start(1, ridx1, g1, sem1)

    def body(h, carry):
        j0 = 2 * h
        drain(j0, ridx0, g0, sem0)

        @pl.when(j0 + 2 < NCHUNK)
        def _():
            start(j0 + 2, ridx0, g0, sem0)

        drain(j0 + 1, ridx1, g1, sem1)

        @pl.when(j0 + 3 < NCHUNK)
        def _():
            start(j0 + 3, ridx1, g1, sem1)

        return carry

    lax.fori_loop(0, NCHUNK // 2, body, 0)


@jax.jit
def _sc_gather(tableRM, idx2):
    mesh = plsc.VectorSubcoreMesh(core_axis_name="c", subcore_axis_name="s")
    fn = pl.kernel(
        _sc_gather_body,
        out_type=jax.ShapeDtypeStruct((BU * EMB_DIM // LANES, LANES),
                                      jnp.float32),
        mesh=mesh,
        scratch_types=[
            pltpu.VMEM((NCHUNK, CHUNK), jnp.int32),
            pltpu.VMEM((CHUNK,), jnp.int32),
            pltpu.VMEM((CHUNK,), jnp.int32),
            pltpu.VMEM((CHUNK, LANES), jnp.float32),
            pltpu.VMEM((CHUNK, LANES), jnp.float32),
            pltpu.VMEM((CHUNK // PACK, LANES), jnp.float32),
            pltpu.SemaphoreType.DMA,
            pltpu.SemaphoreType.DMA,
        ],
        compiler_params=pltpu.CompilerParams(use_tc_tiling_on_sc=False),
    )
    return fn(tableRM, idx2)


CB = 256


def _cmb_body(g_ref, u_ref, v_ref, gn_ref, b_ref, out_ref):
    v = v_ref[...]
    scale = gn_ref[0, 0] * lax.rsqrt(jnp.sum(v * v))
    w = v * scale
    bias = b_ref[...]
    for u in range(U):
        x = u_ref[:, u, :]
        mm = jax.lax.dot_general(
            w, x, dimension_numbers=(((1,), (0,)), ((), ())),
            preferred_element_type=jnp.float32,
        ) + bias
        xg = g_ref[u, 0]
        gt = jnp.concatenate(
            [xg[:, q * EMB_DIM:(q + 1) * EMB_DIM].T for q in range(PACK)],
            axis=1)
        out_ref[u] = jnp.concatenate([gt, mm], axis=0)


@jax.jit
def _tc_combine(gathered4, ufeatT4, V, g2, bcol):
    return pl.pallas_call(
        _cmb_body,
        grid=(B // CB,),
        in_specs=[
            pl.BlockSpec((U, 1, CB * EMB_DIM // LANES, LANES),
                         lambda c: (0, c, 0, 0)),
            pl.BlockSpec((NUM_UFEAT, U, CB), lambda c: (0, 0, c)),
            pl.BlockSpec((EMB_DIM, NUM_UFEAT), lambda c: (0, 0)),
            pl.BlockSpec((1, 1), lambda c: (0, 0), memory_space=pltpu.SMEM),
            pl.BlockSpec((EMB_DIM, 1), lambda c: (0, 0)),
        ],
        out_specs=pl.BlockSpec((U, 2 * EMB_DIM, CB), lambda c: (0, 0, c)),
        out_shape=jax.ShapeDtypeStruct((U, 2 * EMB_DIM, B), jnp.float32),
    )(gathered4, ufeatT4, V, g2, bcol)


NG = B // CB
GR = CB // PACK


def kernel(utype, ufeat, table, V, g, b):
    tableT = table.T
    ufeatT = jnp.transpose(ufeat, (2, 1, 0))
    idxT = utype.T.astype(jnp.int32)
    idx_re = (idxT.reshape(U, NG, PACK, GR)
              .swapaxes(2, 3)
              .reshape(BU // CHUNK, CHUNK))

    tableRM = _tc_transpose(tableT)
    gathered = _sc_gather(tableRM, idx_re)
    g2 = jnp.reshape(g, (1, 1)).astype(jnp.float32)
    bcol = jnp.reshape(b, (EMB_DIM, 1))
    outT = _tc_combine(
        gathered.reshape(U, NG, CB * EMB_DIM // LANES, LANES),
        ufeatT, V, g2, bcol)
    return outT.transpose(2, 0, 1)


# speedup vs baseline: 2.8561x; 1.3951x over previous
import functools

import jax
import jax.numpy as jnp
from jax import lax
from jax.experimental import pallas as pl
from jax.experimental.pallas import tpu as pltpu
from jax.experimental.pallas import tpu_sc as plsc

B, U, NUM_UTYPE, NUM_UFEAT, EMB_DIM = 4096, 50, 1000000, 26, 32
BU = B * U

NC, NS = 2, 16
NW = NC * NS
LANES = 128
PACK = LANES // EMB_DIM


TBLK = 16384
QBLK = TBLK // PACK
NTBLK = pl.cdiv(NUM_UTYPE, TBLK)
RM_PAD_ROWS = NTBLK * QBLK


def _eye(n):
    a = lax.broadcasted_iota(jnp.int32, (n, n), 0)
    b = lax.broadcasted_iota(jnp.int32, (n, n), 1)
    return (a == b).astype(jnp.float32)


def _tr_body(in_ref, out_ref):
    x = in_ref[...]
    y = jax.lax.dot_general(
        x, _eye(EMB_DIM), dimension_numbers=(((0,), (0,)), ((), ())),
        preferred_element_type=jnp.float32,
    )
    parts = [y[q * QBLK:(q + 1) * QBLK, :] for q in range(PACK)]
    out_ref[...] = jnp.concatenate(parts, axis=1)


@jax.jit
def _tc_transpose(tableT):
    return pl.pallas_call(
        _tr_body,
        grid=(NTBLK,),
        in_specs=[pl.BlockSpec((EMB_DIM, TBLK), lambda k: (0, k))],
        out_specs=pl.BlockSpec((QBLK, LANES), lambda k: (k, 0)),
        out_shape=jax.ShapeDtypeStruct((RM_PAD_ROWS, LANES), jnp.float32),
    )(tableT)


ROWS_PER_W = BU // NW
CHUNK = 128
NCHUNK = ROWS_PER_W // CHUNK


def _sc_gather_body(table_hbm, idx_hbm, out_hbm, idx_v,
                    ridx0, ridx1, g0, g1, sem0, sem1):
    wid = lax.axis_index("s") * NC + lax.axis_index("c")
    base = wid * ROWS_PER_W
    pltpu.sync_copy(idx_hbm.at[pl.ds(wid * NCHUNK, NCHUNK)], idx_v)

    def start(j, ridx, buf, sem):
        for m in range(CHUNK // 16):
            iv = idx_v[j, pl.ds(m * 16, 16)]
            r32 = (lax.shift_left(lax.shift_right_logical(iv, 14), 14)
                   | lax.shift_left(iv & 4095, 2)
                   | (lax.shift_right_logical(iv, 12) & 3))
            ridx[pl.ds(m * 16, 16)] = r32
        pltpu.make_async_copy(table_hbm.at[ridx], buf, sem).start()

    def drain(j, ridx, buf, sem):
        pltpu.make_async_copy(table_hbm.at[ridx], buf, sem).wait()
        pltpu.sync_copy(buf, out_hbm.at[pl.ds(base + j * CHUNK, CHUNK)])

    start(0, ridx0, g0, sem0)
    start(1, ridx1, g1, sem1)

    def body(h, carry):
        j0 = 2 * h
        drain(j0, ridx0, g0, sem0)

        @pl.when(j0 + 2 < NCHUNK)
        def _():
            start(j0 + 2, ridx0, g0, sem0)

        drain(j0 + 1, ridx1, g1, sem1)

        @pl.when(j0 + 3 < NCHUNK)
        def _():
            start(j0 + 3, ridx1, g1, sem1)

        return carry

    lax.fori_loop(0, NCHUNK // 2, body, 0)


@jax.jit
def _sc_gather(tableRM, idx2):
    mesh = plsc.VectorSubcoreMesh(core_axis_name="c", subcore_axis_name="s")
    fn = pl.kernel(
        _sc_gather_body,
        out_type=jax.ShapeDtypeStruct((BU, EMB_DIM), jnp.float32),
        mesh=mesh,
        scratch_types=[
            pltpu.VMEM((NCHUNK, CHUNK), jnp.int32),
            pltpu.VMEM((CHUNK,), jnp.int32),
            pltpu.VMEM((CHUNK,), jnp.int32),
            pltpu.VMEM((CHUNK, EMB_DIM), jnp.float32),
            pltpu.VMEM((CHUNK, EMB_DIM), jnp.float32),
            pltpu.SemaphoreType.DMA,
            pltpu.SemaphoreType.DMA,
        ],
        compiler_params=pltpu.CompilerParams(use_tc_tiling_on_sc=False),
    )
    return fn(tableRM.reshape(RM_PAD_ROWS * PACK, EMB_DIM), idx2)


CB = 256
GR = CB // PACK
NG = B // CB


def _cmb_body(g_ref, u_ref, v_ref, gn_ref, b_ref, out_ref):
    v = v_ref[...]
    scale = gn_ref[0, 0] * lax.rsqrt(jnp.sum(v * v))
    w = v * scale
    bias = b_ref[...]
    eye = _eye(GR)
    for u in range(U):
        x = u_ref[:, u, :]
        mm = jax.lax.dot_general(
            w, x, dimension_numbers=(((1,), (0,)), ((), ())),
            preferred_element_type=jnp.float32,
        ) + bias
        xg = g_ref[u, 0]
        xgT = jax.lax.dot_general(
            xg, eye, dimension_numbers=(((0,), (0,)), ((), ())),
            preferred_element_type=jnp.float32,
        )
        gt = jnp.concatenate(
            [xgT[q * EMB_DIM:(q + 1) * EMB_DIM, :] for q in range(PACK)],
            axis=1)
        out_ref[u] = jnp.concatenate([gt, mm], axis=0)


@jax.jit
def _tc_combine(gathered4, ufeatT, V, g2, bcol):
    return pl.pallas_call(
        _cmb_body,
        grid=(B // CB,),
        in_specs=[
            pl.BlockSpec((U, 1, GR, LANES), lambda c: (0, c, 0, 0)),
            pl.BlockSpec((NUM_UFEAT, U, CB), lambda c: (0, 0, c)),
            pl.BlockSpec((EMB_DIM, NUM_UFEAT), lambda c: (0, 0)),
            pl.BlockSpec((1, 1), lambda c: (0, 0), memory_space=pltpu.SMEM),
            pl.BlockSpec((EMB_DIM, 1), lambda c: (0, 0)),
        ],
        out_specs=pl.BlockSpec((U, 2 * EMB_DIM, CB), lambda c: (0, 0, c)),
        out_shape=jax.ShapeDtypeStruct((U, 2 * EMB_DIM, B), jnp.float32),
    )(gathered4, ufeatT, V, g2, bcol)


def kernel(utype, ufeat, table, V, g, b):
    tableT = table.T
    ufeatT = jnp.transpose(ufeat, (2, 1, 0))
    idxT = utype.T.astype(jnp.int32)
    idx_re = (idxT.reshape(U, NG, PACK, GR)
              .swapaxes(2, 3)
              .reshape(BU // CHUNK, CHUNK))

    tableRM = _tc_transpose(tableT)
    gathered = _sc_gather(tableRM, idx_re)
    g2 = jnp.reshape(g, (1, 1)).astype(jnp.float32)
    bcol = jnp.reshape(b, (EMB_DIM, 1))
    outT = _tc_combine(
        gathered.reshape(U, NG, GR, LANES), ufeatT, V, g2, bcol)
    return outT.transpose(2, 0, 1)


# speedup vs baseline: 4.6893x; 1.6419x over previous
import functools

import jax
import jax.numpy as jnp
from jax import lax
from jax.experimental import pallas as pl
from jax.experimental.pallas import tpu as pltpu
from jax.experimental.pallas import tpu_sc as plsc

B, U, NUM_UTYPE, NUM_UFEAT, EMB_DIM = 4096, 50, 1000000, 26, 32
BU = B * U

NC, NS = 2, 16
NW = NC * NS
LANES = 128
PACK = LANES // EMB_DIM


TBLK = 16384
QBLK = TBLK // PACK
NTBLK = pl.cdiv(NUM_UTYPE, TBLK)
RM_PAD_ROWS = NTBLK * QBLK


def _eye(n):
    a = lax.broadcasted_iota(jnp.int32, (n, n), 0)
    b = lax.broadcasted_iota(jnp.int32, (n, n), 1)
    return (a == b).astype(jnp.float32)


def _tr_body(in_ref, out_ref):
    x = in_ref[...]
    xs = jnp.concatenate(
        [x[:, q * QBLK:(q + 1) * QBLK] for q in range(PACK)], axis=0)
    out_ref[...] = jax.lax.dot_general(
        xs, _eye(LANES), dimension_numbers=(((0,), (0,)), ((), ())),
        preferred_element_type=jnp.float32,
    )


@jax.jit
def _tc_transpose(tableT):
    return pl.pallas_call(
        _tr_body,
        grid=(NTBLK,),
        in_specs=[pl.BlockSpec((EMB_DIM, TBLK), lambda k: (0, k))],
        out_specs=pl.BlockSpec((QBLK, LANES), lambda k: (k, 0)),
        out_shape=jax.ShapeDtypeStruct((RM_PAD_ROWS, LANES), jnp.float32),
    )(tableT)


ROWS_PER_W = BU // NW
CHUNK = 128
NCHUNK = ROWS_PER_W // CHUNK


def _sc_gather_body(table_hbm, idx_hbm, out_hbm, idx_v,
                    ridx0, ridx1, g0, g1, sem0, sem1):
    wid = lax.axis_index("s") * NC + lax.axis_index("c")
    base = wid * ROWS_PER_W
    pltpu.sync_copy(idx_hbm.at[pl.ds(wid * NCHUNK, NCHUNK)], idx_v)

    def start(j, ridx, buf, sem):
        for m in range(CHUNK // 16):
            iv = idx_v[j, pl.ds(m * 16, 16)]
            r32 = (lax.shift_left(lax.shift_right_logical(iv, 14), 14)
                   | lax.shift_left(iv & 4095, 2)
                   | (lax.shift_right_logical(iv, 12) & 3))
            ridx[pl.ds(m * 16, 16)] = r32
        pltpu.make_async_copy(table_hbm.at[ridx], buf, sem).start()

    def drain(j, ridx, buf, sem):
        pltpu.make_async_copy(table_hbm.at[ridx], buf, sem).wait()
        pltpu.sync_copy(buf, out_hbm.at[pl.ds(base + j * CHUNK, CHUNK)])

    start(0, ridx0, g0, sem0)
    start(1, ridx1, g1, sem1)

    def body(h, carry):
        j0 = 2 * h
        drain(j0, ridx0, g0, sem0)

        @pl.when(j0 + 2 < NCHUNK)
        def _():
            start(j0 + 2, ridx0, g0, sem0)

        drain(j0 + 1, ridx1, g1, sem1)

        @pl.when(j0 + 3 < NCHUNK)
        def _():
            start(j0 + 3, ridx1, g1, sem1)

        return carry

    lax.fori_loop(0, NCHUNK // 2, body, 0)


@jax.jit
def _sc_gather(tableRM, idx2):
    mesh = plsc.VectorSubcoreMesh(core_axis_name="c", subcore_axis_name="s")
    fn = pl.kernel(
        _sc_gather_body,
        out_type=jax.ShapeDtypeStruct((BU, EMB_DIM), jnp.float32),
        mesh=mesh,
        scratch_types=[
            pltpu.VMEM((NCHUNK, CHUNK), jnp.int32),
            pltpu.VMEM((CHUNK,), jnp.int32),
            pltpu.VMEM((CHUNK,), jnp.int32),
            pltpu.VMEM((CHUNK, EMB_DIM), jnp.float32),
            pltpu.VMEM((CHUNK, EMB_DIM), jnp.float32),
            pltpu.SemaphoreType.DMA,
            pltpu.SemaphoreType.DMA,
        ],
        compiler_params=pltpu.CompilerParams(use_tc_tiling_on_sc=False),
    )
    return fn(tableRM.reshape(RM_PAD_ROWS * PACK, EMB_DIM), idx2)


CB = 256
GR = CB // PACK
NG = B // CB


def _cmb_body(g_ref, u_ref, v_ref, gn_ref, b_ref, out_ref):
    v = v_ref[...]
    scale = gn_ref[0, 0] * lax.rsqrt(jnp.sum(v * v))
    w = v * scale
    bias = b_ref[...]
    eye = _eye(GR)
    for u in range(U):
        x = u_ref[:, u, :]
        mm = jax.lax.dot_general(
            w, x, dimension_numbers=(((1,), (0,)), ((), ())),
            preferred_element_type=jnp.float32,
        ) + bias
        xg = g_ref[u, 0]
        xgT = jax.lax.dot_general(
            xg, eye, dimension_numbers=(((0,), (0,)), ((), ())),
            preferred_element_type=jnp.float32,
        )
        gt = jnp.concatenate(
            [xgT[q * EMB_DIM:(q + 1) * EMB_DIM, :] for q in range(PACK)],
            axis=1)
        out_ref[u] = jnp.concatenate([gt, mm], axis=0)


@jax.jit
def _tc_combine(gathered4, ufeatT, V, g2, bcol):
    return pl.pallas_call(
        _cmb_body,
        grid=(B // CB,),
        in_specs=[
            pl.BlockSpec((U, 1, GR, LANES), lambda c: (0, c, 0, 0)),
            pl.BlockSpec((NUM_UFEAT, U, CB), lambda c: (0, 0, c)),
            pl.BlockSpec((EMB_DIM, NUM_UFEAT), lambda c: (0, 0)),
            pl.BlockSpec((1, 1), lambda c: (0, 0), memory_space=pltpu.SMEM),
            pl.BlockSpec((EMB_DIM, 1), lambda c: (0, 0)),
        ],
        out_specs=pl.BlockSpec((U, 2 * EMB_DIM, CB), lambda c: (0, 0, c)),
        out_shape=jax.ShapeDtypeStruct((U, 2 * EMB_DIM, B), jnp.float32),
    )(gathered4, ufeatT, V, g2, bcol)


def kernel(utype, ufeat, table, V, g, b):
    tableT = table.T
    ufeatT = jnp.transpose(ufeat, (2, 1, 0))
    idxT = utype.T.astype(jnp.int32)
    idx_re = (idxT.reshape(U, NG, PACK, GR)
              .swapaxes(2, 3)
              .reshape(BU // CHUNK, CHUNK))

    tableRM = _tc_transpose(tableT)
    gathered = _sc_gather(tableRM, idx_re)
    g2 = jnp.reshape(g, (1, 1)).astype(jnp.float32)
    bcol = jnp.reshape(b, (EMB_DIM, 1))
    outT = _tc_combine(
        gathered.reshape(U, NG, GR, LANES), ufeatT, V, g2, bcol)
    return outT.transpose(2, 0, 1)
